# baseline (speedup 1.0000x reference)
import functools

import jax
import jax.numpy as jnp
from jax import lax
from jax.experimental import pallas as pl
from jax.experimental.pallas import tpu as pltpu
from jax.experimental.pallas import tpu_sc as plsc

_PAD = 128


def _l1_body(e_ref, w1_ref, b1_ref, o_ref):
    h = jnp.dot(e_ref[...], w1_ref[...], preferred_element_type=jnp.float32)
    o_ref[...] = jnp.maximum(h + b1_ref[...], 0.0)


def _layer1(esm, W1p, b1p, row_block):
    g, d = esm.shape
    grid = (g + row_block - 1) // row_block
    return pl.pallas_call(
        _l1_body,
        grid=(grid,),
        in_specs=[
            pl.BlockSpec((row_block, d), lambda i: (i, 0)),
            pl.BlockSpec((d, _PAD), lambda i: (0, 0)),
            pl.BlockSpec((1, _PAD), lambda i: (0, 0)),
        ],
        out_specs=pl.BlockSpec((row_block, _PAD), lambda i: (i, 0)),
        out_shape=jax.ShapeDtypeStruct((g, _PAD), jnp.float32),
    )(esm, W1p, b1p)


def _gather_body(n_chunks, ch, nc, idx_hbm, tbl_hbm, out_hbm,
                 idx_v, rows_v, sem):
    wid = lax.axis_index("s") * nc + lax.axis_index("c")
    base = wid * (n_chunks * ch)
    for i in range(n_chunks):
        off = base + i * ch
        pltpu.sync_copy(idx_hbm.at[pl.ds(off, ch)], idx_v)
        pltpu.async_copy(tbl_hbm.at[idx_v], rows_v, sem).wait()
        pltpu.sync_copy(rows_v, out_hbm.at[pl.ds(off, ch)])


def _sc_gather(table, idx_flat):
    _, hid = table.shape
    bs = idx_flat.shape[0]
    info = plsc.get_sparse_core_info()
    nc, ns = info.num_cores, info.num_subcores
    nw = nc * ns
    per_w = bs // nw
    ch = 640
    n_chunks = per_w // ch
    body = functools.partial(_gather_body, n_chunks, ch, nc)
    kern = pl.kernel(
        body,
        out_type=jax.ShapeDtypeStruct((bs, hid), jnp.float32),
        mesh=plsc.VectorSubcoreMesh(core_axis_name="c", subcore_axis_name="s"),
        scratch_types=[
            pltpu.VMEM((ch,), jnp.int32),
            pltpu.VMEM((ch, hid), jnp.float32),
            pltpu.SemaphoreType.DMA,
        ],
        compiler_params=pltpu.CompilerParams(use_tc_tiling_on_sc=True),
    )
    return kern(idx_flat, table)


def _l2_body(s, hid, x_ref, w2_ref, b2_ref, o_ref):
    y = jnp.dot(x_ref[...], w2_ref[...], preferred_element_type=jnp.float32)
    y = y + b2_ref[...]
    o_ref[...] = y.reshape(o_ref.shape)


def _layer2(rows, W2p, b2, b, s, hid, b_block):
    bs = rows.shape[0]
    grid = b // b_block
    body = functools.partial(_l2_body, s, hid)
    return pl.pallas_call(
        body,
        grid=(grid,),
        in_specs=[
            pl.BlockSpec((b_block * s, _PAD), lambda i: (i, 0)),
            pl.BlockSpec((_PAD, hid), lambda i: (0, 0)),
            pl.BlockSpec((1, hid), lambda i: (0, 0)),
        ],
        out_specs=pl.BlockSpec((b_block, s, hid), lambda i: (i, 0, 0)),
        out_shape=jax.ShapeDtypeStruct((b, s, hid), jnp.float32),
    )(rows, W2p, b2.reshape(1, hid))


def kernel(pert_esm_indices, esm_embeddings, W1, b1, W2, b2):
    idx = pert_esm_indices
    if idx.shape[-1] == 1:
        idx = jnp.squeeze(idx, axis=-1)
    b, s = idx.shape
    d, hid = W1.shape
    W1p = jnp.pad(W1, ((0, 0), (0, _PAD - hid)))
    b1p = jnp.pad(b1, (0, _PAD - hid)).reshape(1, _PAD)
    W2p = jnp.pad(W2, ((0, _PAD - hid), (0, 0)))
    table = _layer1(esm_embeddings, W1p, b1p, row_block=800)
    idx_flat = idx.reshape(-1).astype(jnp.int32)
    rows = _sc_gather(table, idx_flat)
    return _layer2(rows, W2p, b2, b, s, hid, b_block=128)


# speedup vs baseline: 1.5073x; 1.5073x over previous
import functools

import jax
import jax.numpy as jnp
from jax import lax
from jax.experimental import pallas as pl
from jax.experimental.pallas import tpu as pltpu
from jax.experimental.pallas import tpu_sc as plsc

_PAD = 128


def _l1_body(hid, e_ref, w1_ref, b1_ref, o_ref):
    h = jnp.dot(e_ref[...], w1_ref[...], preferred_element_type=jnp.float32)
    o_ref[:, :hid] = jnp.maximum(h + b1_ref[...], 0.0)
    o_ref[:, hid:] = jnp.zeros((o_ref.shape[0], _PAD - hid), jnp.float32)


def _layer1(esm, W1, b1, row_block):
    g, d = esm.shape
    hid = W1.shape[1]
    grid = (g + row_block - 1) // row_block
    return pl.pallas_call(
        functools.partial(_l1_body, hid),
        grid=(grid,),
        in_specs=[
            pl.BlockSpec((row_block, d), lambda i: (i, 0)),
            pl.BlockSpec((d, hid), lambda i: (0, 0)),
            pl.BlockSpec((1, hid), lambda i: (0, 0)),
        ],
        out_specs=pl.BlockSpec((row_block, _PAD), lambda i: (i, 0)),
        out_shape=jax.ShapeDtypeStruct((g, _PAD), jnp.float32),
    )(esm, W1, b1.reshape(1, hid))


def _gather_body(n_chunks, ch, nc, idx_hbm, tbl_hbm, out_hbm,
                 idx_v, r0, r1, s0, s1):
    wid = lax.axis_index("s") * nc + lax.axis_index("c")
    per_w = n_chunks * ch
    base = wid * per_w
    pltpu.sync_copy(idx_hbm.at[pl.ds(base, per_w)], idx_v)
    bufs, sems, descs = (r0, r1), (s0, s1), [None, None]
    descs[0] = pltpu.async_copy(tbl_hbm.at[idx_v.at[pl.ds(0, ch)]], r0, s0)
    for i in range(1, n_chunks):
        descs[i % 2] = pltpu.async_copy(
            tbl_hbm.at[idx_v.at[pl.ds(i * ch, ch)]], bufs[i % 2], sems[i % 2])
        descs[(i - 1) % 2].wait()
        pltpu.sync_copy(bufs[(i - 1) % 2],
                        out_hbm.at[pl.ds(base + (i - 1) * ch, ch)])
    last = n_chunks - 1
    descs[last % 2].wait()
    pltpu.sync_copy(bufs[last % 2], out_hbm.at[pl.ds(base + last * ch, ch)])


def _sc_gather(table, idx_flat):
    _, hid = table.shape
    bs = idx_flat.shape[0]
    info = plsc.get_sparse_core_info()
    nc, ns = info.num_cores, info.num_subcores
    nw = nc * ns
    per_w = bs // nw
    ch = 320
    n_chunks = per_w // ch
    body = functools.partial(_gather_body, n_chunks, ch, nc)
    kern = pl.kernel(
        body,
        out_type=jax.ShapeDtypeStruct((bs, hid), jnp.float32),
        mesh=plsc.VectorSubcoreMesh(core_axis_name="c", subcore_axis_name="s"),
        scratch_types=[
            pltpu.VMEM((per_w,), jnp.int32),
            pltpu.VMEM((ch, hid), jnp.float32),
            pltpu.VMEM((ch, hid), jnp.float32),
            pltpu.SemaphoreType.DMA,
            pltpu.SemaphoreType.DMA,
        ],
        compiler_params=pltpu.CompilerParams(use_tc_tiling_on_sc=True),
    )
    return kern(idx_flat, table)


def _l2_body(x_ref, w2t_ref, b2_ref, o_ref):
    y = lax.dot_general(
        w2t_ref[...], x_ref[0],
        dimension_numbers=(((1,), (1,)), ((), ())),
        preferred_element_type=jnp.float32,
    )
    o_ref[0] = y + b2_ref[...]


def _layer2_t(rows3, W2pT, b2, s, b, hid, b_block):
    return pl.pallas_call(
        _l2_body,
        grid=(s, b // b_block),
        in_specs=[
            pl.BlockSpec((1, b_block, _PAD), lambda i, j: (i, j, 0)),
            pl.BlockSpec((hid, _PAD), lambda i, j: (0, 0)),
            pl.BlockSpec((hid, 1), lambda i, j: (0, 0)),
        ],
        out_specs=pl.BlockSpec((1, hid, b_block), lambda i, j: (i, 0, j)),
        out_shape=jax.ShapeDtypeStruct((s, hid, b), jnp.float32),
    )(rows3, W2pT, b2.reshape(hid, 1))


def kernel(pert_esm_indices, esm_embeddings, W1, b1, W2, b2):
    idx = pert_esm_indices
    if idx.shape[-1] == 1:
        idx = jnp.squeeze(idx, axis=-1)
    b, s = idx.shape
    d, hid = W1.shape
    W2pT = jnp.pad(W2, ((0, _PAD - hid), (0, 0))).T
    table = _layer1(esm_embeddings, W1, b1, row_block=2000)
    idx_flat_t = idx.T.reshape(-1).astype(jnp.int32)
    rows = _sc_gather(table, idx_flat_t)
    rows3 = rows.reshape(s, b, _PAD)
    out_t = _layer2_t(rows3, W2pT, b2, s, b, hid, b_block=4096)
    return jnp.transpose(out_t, (2, 0, 1))
